# baseline (speedup 1.0000x reference)
import functools
import numpy as np
import jax
import jax.numpy as jnp
from jax.experimental import pallas as pl

_HIGH = jax.lax.Precision.HIGHEST


def _body(score_ref, hid_ref, w1_ref, w2_ref, outh_ref, sel_ref, pidx_ref,
          *, C, S, K, H):
    f32 = jnp.float32
    score = score_ref[0]
    iota_cs = jax.lax.broadcasted_iota(jnp.int32, (C, S), 1)

    def step(_, work):
        m = jnp.max(work, axis=1, keepdims=True)
        ismax = work == m
        fidx = jnp.min(jnp.where(ismax, iota_cs, S), axis=1, keepdims=True)
        pick = iota_cs == fidx
        return jnp.where(pick, -jnp.inf, work)

    work = jax.lax.fori_loop(0, K, step, score)
    selm = work != score

    new_score = jnp.where(selm, score, score * f32(0.7))
    pw = jnp.mean(new_score, axis=0, keepdims=True)
    msum = jnp.sum(new_score, axis=0, keepdims=True)
    thr = jnp.mean(msum)
    mvals = jnp.where(msum > thr, pw, f32(0.0))
    iota_s = jax.lax.broadcasted_iota(jnp.int32, (1, S), 1)
    mx = jnp.max(mvals)
    ridx = jnp.min(jnp.where(mvals == mx, iota_s, S))

    ai = ridx // H
    aj = ridx - ai * H
    ii = iota_s // H
    jj = iota_s - ii * H
    inv = f32(1.0 / H)
    rel_i = (ii.astype(f32) - ai.astype(f32)) * inv
    rel_j = (jj.astype(f32) - aj.astype(f32)) * inv
    rdist = jnp.sqrt(rel_i * rel_i + rel_j * rel_j)
    rang = (jnp.arctan2(rel_j, rel_i) * f32(1.0 / np.pi) + f32(1.0)) * f32(0.5)
    a = jnp.sum(pw * rdist)
    b = jnp.sum(pw * rang)
    pp = jnp.sum(jnp.where(pw > 0, pw * pw, f32(0.0)))
    pn = jnp.sum(jnp.where(pw < 0, pw * pw, f32(0.0)))
    u = a * w1_ref[0:1, :] + b * w1_ref[1:2, :]
    wvec = jnp.where(u > 0, pp, pn) * u
    gv = jnp.dot(wvec, w2_ref[:, :], precision=_HIGH,
                 preferred_element_type=f32)
    pw_a = jnp.sum(jnp.where(iota_s == ridx, pw, f32(0.0)))
    z = pw_a * gv
    add_vec = jnp.where(z >= 0, z, f32(0.2) * z)

    outh_ref[0] = hid_ref[0]
    outh_ref[0, 0:1, :] = hid_ref[0, 0:1, :] + add_vec

    count_row = jnp.sum(selm.astype(f32), axis=0, keepdims=True)
    p_col = jax.lax.broadcasted_iota(jnp.int32, (S, S), 0)
    q_row = jax.lax.broadcasted_iota(jnp.int32, (S, S), 1)
    pi = p_col // H
    pj = p_col - pi * H
    qi = q_row // H
    qj = q_row - qi * H
    di = jnp.abs(pi - qi)
    dj = jnp.abs(pj - qj)
    t1 = jnp.where(di == 0, f32(2.0), jnp.where(di == 1, f32(1.0), f32(0.0)))
    t2 = jnp.where(dj == 0, f32(2.0), jnp.where(dj == 1, f32(1.0), f32(0.0)))
    m_band = t1 * t2
    y_row = jnp.dot(count_row, m_band, precision=_HIGH,
                    preferred_element_type=f32)
    dy = jnp.where(p_col == q_row, y_row, f32(0.0))
    y_col = jnp.dot(dy, jnp.ones((S, 1), f32), precision=_HIGH,
                    preferred_element_type=f32)
    iota_col = jax.lax.broadcasted_iota(jnp.int32, (S, 1), 0)
    key_row = y_row * f32(S) + (S - 1 - iota_s).astype(f32)
    key_col = y_col * f32(S) + (S - 1 - iota_col).astype(f32)
    rank_col = jnp.sum((key_row > key_col).astype(f32), axis=1, keepdims=True)
    rank_row = jnp.sum((key_col > key_row).astype(f32), axis=0, keepdims=True)

    pos_b = jnp.where(rank_col == iota_s.astype(f32),
                      (iota_col + 1).astype(f32), f32(0.0))
    pos_row = jnp.sum(pos_b, axis=0, keepdims=True)
    pidx_ref[0] = pos_row[:, :K].astype(jnp.int32)

    iota_col_k = jax.lax.broadcasted_iota(jnp.int32, (K, 1), 0)
    onehot = (rank_row == iota_col_k.astype(f32)).astype(f32)
    sel_ref[0] = jnp.dot(onehot, hid_ref[0, 1:, :], precision=_HIGH,
                         preferred_element_type=f32)


def kernel(hidden_states, x, contribution, W1, W2):
    B, C = x.shape[0], x.shape[1]
    S = x.shape[3] - 1
    K = 84
    H = int(np.sqrt(S))
    T, HD = hidden_states.shape[1], hidden_states.shape[2]
    score = x[:, :, 0, 1:]

    body = functools.partial(_body, C=C, S=S, K=K, H=H)
    outh, sel, pidx = pl.pallas_call(
        body,
        grid=(B,),
        in_specs=[
            pl.BlockSpec((1, C, S), lambda b: (b, 0, 0)),
            pl.BlockSpec((1, T, HD), lambda b: (b, 0, 0)),
            pl.BlockSpec((2, 512), lambda b: (0, 0)),
            pl.BlockSpec((512, HD), lambda b: (0, 0)),
        ],
        out_specs=[
            pl.BlockSpec((1, T, HD), lambda b: (b, 0, 0)),
            pl.BlockSpec((1, K, HD), lambda b: (b, 0, 0)),
            pl.BlockSpec((1, 1, K), lambda b: (b, 0, 0)),
        ],
        out_shape=[
            jax.ShapeDtypeStruct((B, T, HD), jnp.float32),
            jax.ShapeDtypeStruct((B, K, HD), jnp.float32),
            jax.ShapeDtypeStruct((B, 1, K), jnp.int32),
        ],
    )(score, hidden_states, W1, W2)
    return outh, sel, pidx.reshape(B, K)


# speedup vs baseline: 2.1908x; 2.1908x over previous
import functools
import numpy as np
import jax
import jax.numpy as jnp
from jax.experimental import pallas as pl

_HIGH = jax.lax.Precision.HIGHEST


def _body(score_ref, hid_ref, w1_ref, w2_ref, outh_ref, sel_ref, pidx_ref,
          *, C, S, K, H):
    f32 = jnp.float32
    score = score_ref[0]

    sb = jax.lax.bitcast_convert_type(score, jnp.int32)
    keys = sb ^ (jax.lax.shift_right_arithmetic(sb, 31) &
                 jnp.int32(0x7FFFFFFF))
    rmin = jnp.min(keys, axis=1, keepdims=True)
    rmax = jnp.max(keys, axis=1, keepdims=True)
    g0 = jnp.sum((keys >= 0).astype(jnp.int32), axis=1, keepdims=True)
    pos_side = g0 >= K
    lo0 = jnp.where(pos_side, jnp.maximum(rmin, 0), rmin)
    hi0 = jnp.where(pos_side, rmax, jnp.minimum(rmax, -1))

    def bstep(_, carry):
        lo, hi = carry
        mid = lo + jax.lax.shift_right_arithmetic(hi - lo, 1)
        cnt = jnp.sum((keys > mid).astype(jnp.int32), axis=1, keepdims=True)
        ge = cnt >= K
        return (jnp.where(ge, mid + 1, lo), jnp.where(ge, hi, mid))

    t, _ = jax.lax.fori_loop(0, 31, bstep, (lo0, hi0))
    gt = keys > t
    g = jnp.sum(gt.astype(jnp.int32), axis=1, keepdims=True)
    tie = keys == t
    p_tri = jax.lax.broadcasted_iota(jnp.int32, (S, S), 0)
    q_tri = jax.lax.broadcasted_iota(jnp.int32, (S, S), 1)
    tri = (p_tri <= q_tri).astype(jnp.float32)
    prefix = jnp.dot(tie.astype(jnp.float32), tri,
                     preferred_element_type=jnp.float32)
    selm = jnp.logical_or(gt, jnp.logical_and(
        tie, prefix <= (K - g).astype(jnp.float32)))

    new_score = jnp.where(selm, score, score * f32(0.7))
    pw = jnp.mean(new_score, axis=0, keepdims=True)
    msum = jnp.sum(new_score, axis=0, keepdims=True)
    thr = jnp.mean(msum)
    mvals = jnp.where(msum > thr, pw, f32(0.0))
    iota_s = jax.lax.broadcasted_iota(jnp.int32, (1, S), 1)
    mx = jnp.max(mvals)
    ridx = jnp.min(jnp.where(mvals == mx, iota_s, S))

    ai = ridx // H
    aj = ridx - ai * H
    ii = iota_s // H
    jj = iota_s - ii * H
    inv = f32(1.0 / H)
    rel_i = (ii.astype(f32) - ai.astype(f32)) * inv
    rel_j = (jj.astype(f32) - aj.astype(f32)) * inv
    rdist = jnp.sqrt(rel_i * rel_i + rel_j * rel_j)
    rang = (jnp.arctan2(rel_j, rel_i) * f32(1.0 / np.pi) + f32(1.0)) * f32(0.5)
    a = jnp.sum(pw * rdist)
    b = jnp.sum(pw * rang)
    pp = jnp.sum(jnp.where(pw > 0, pw * pw, f32(0.0)))
    pn = jnp.sum(jnp.where(pw < 0, pw * pw, f32(0.0)))
    u = a * w1_ref[0:1, :] + b * w1_ref[1:2, :]
    wvec = jnp.where(u > 0, pp, pn) * u
    gv = jnp.dot(wvec, w2_ref[:, :], precision=_HIGH,
                 preferred_element_type=f32)
    pw_a = jnp.sum(jnp.where(iota_s == ridx, pw, f32(0.0)))
    z = pw_a * gv
    add_vec = jnp.where(z >= 0, z, f32(0.2) * z)

    outh_ref[0] = hid_ref[0]
    outh_ref[0, 0:1, :] = hid_ref[0, 0:1, :] + add_vec

    count_row = jnp.sum(selm.astype(f32), axis=0, keepdims=True)
    p_col = jax.lax.broadcasted_iota(jnp.int32, (S, S), 0)
    q_row = jax.lax.broadcasted_iota(jnp.int32, (S, S), 1)
    pi = p_col // H
    pj = p_col - pi * H
    qi = q_row // H
    qj = q_row - qi * H
    di = jnp.abs(pi - qi)
    dj = jnp.abs(pj - qj)
    t1 = jnp.where(di == 0, f32(2.0), jnp.where(di == 1, f32(1.0), f32(0.0)))
    t2 = jnp.where(dj == 0, f32(2.0), jnp.where(dj == 1, f32(1.0), f32(0.0)))
    m_band = t1 * t2
    y_row = jnp.dot(count_row, m_band, precision=_HIGH,
                    preferred_element_type=f32)
    dy = jnp.where(p_col == q_row, y_row, f32(0.0))
    y_col = jnp.dot(dy, jnp.ones((S, 1), f32), precision=_HIGH,
                    preferred_element_type=f32)
    iota_col = jax.lax.broadcasted_iota(jnp.int32, (S, 1), 0)
    key_row = y_row * f32(S) + (S - 1 - iota_s).astype(f32)
    key_col = y_col * f32(S) + (S - 1 - iota_col).astype(f32)
    rank_col = jnp.sum((key_row > key_col).astype(f32), axis=1, keepdims=True)
    rank_row = jnp.sum((key_col > key_row).astype(f32), axis=0, keepdims=True)

    pos_b = jnp.where(rank_col == iota_s.astype(f32),
                      (iota_col + 1).astype(f32), f32(0.0))
    pos_row = jnp.sum(pos_b, axis=0, keepdims=True)
    pidx_ref[0] = pos_row[:, :K].astype(jnp.int32)

    iota_col_k = jax.lax.broadcasted_iota(jnp.int32, (K, 1), 0)
    onehot = (rank_row == iota_col_k.astype(f32)).astype(f32)
    sel_ref[0] = jnp.dot(onehot, hid_ref[0, 1:, :], precision=_HIGH,
                         preferred_element_type=f32)


def kernel(hidden_states, x, contribution, W1, W2):
    B, C = x.shape[0], x.shape[1]
    S = x.shape[3] - 1
    K = 84
    H = int(np.sqrt(S))
    T, HD = hidden_states.shape[1], hidden_states.shape[2]
    score = x[:, :, 0, 1:]

    body = functools.partial(_body, C=C, S=S, K=K, H=H)
    outh, sel, pidx = pl.pallas_call(
        body,
        grid=(B,),
        in_specs=[
            pl.BlockSpec((1, C, S), lambda b: (b, 0, 0)),
            pl.BlockSpec((1, T, HD), lambda b: (b, 0, 0)),
            pl.BlockSpec((2, 512), lambda b: (0, 0)),
            pl.BlockSpec((512, HD), lambda b: (0, 0)),
        ],
        out_specs=[
            pl.BlockSpec((1, T, HD), lambda b: (b, 0, 0)),
            pl.BlockSpec((1, K, HD), lambda b: (b, 0, 0)),
            pl.BlockSpec((1, 1, K), lambda b: (b, 0, 0)),
        ],
        out_shape=[
            jax.ShapeDtypeStruct((B, T, HD), jnp.float32),
            jax.ShapeDtypeStruct((B, K, HD), jnp.float32),
            jax.ShapeDtypeStruct((B, 1, K), jnp.int32),
        ],
    )(score, hidden_states, W1, W2)
    return outh, sel, pidx.reshape(B, K)


# speedup vs baseline: 3.4703x; 1.5840x over previous
import functools
import numpy as np
import jax
import jax.numpy as jnp
from jax.experimental import pallas as pl

_HIGH = jax.lax.Precision.HIGHEST


def _select_body(score_ref, w1_ref, w2_ref, add_ref, pidx_ref, oh_ref,
                 *, B, C, S, K, H):
    f32 = jnp.float32
    R = B * C
    sflat = score_ref[...]

    sb = jax.lax.bitcast_convert_type(sflat, jnp.int32)
    keys = sb ^ (jax.lax.shift_right_arithmetic(sb, 31) &
                 jnp.int32(0x7FFFFFFF))
    rmin = jnp.min(keys, axis=1, keepdims=True)
    rmax = jnp.max(keys, axis=1, keepdims=True)
    g0 = jnp.sum((keys >= 0).astype(jnp.int32), axis=1, keepdims=True)
    pos_side = g0 >= K
    lo0 = jnp.where(pos_side, jnp.maximum(rmin, 0), rmin)
    hi0 = jnp.where(pos_side, rmax, jnp.minimum(rmax, -1))

    def bstep(_, carry):
        lo, hi = carry
        mid = lo + jax.lax.shift_right_arithmetic(hi - lo, 1)
        cnt = jnp.sum((keys > mid).astype(jnp.int32), axis=1, keepdims=True)
        ge = cnt >= K
        return (jnp.where(ge, mid + 1, lo), jnp.where(ge, hi, mid))

    t, _ = jax.lax.fori_loop(0, 31, bstep, (lo0, hi0))
    gt = keys > t
    g = jnp.sum(gt.astype(jnp.int32), axis=1, keepdims=True)
    tie = keys == t
    icol = jax.lax.broadcasted_iota(jnp.int32, (S, 1), 0)
    irow = jax.lax.broadcasted_iota(jnp.int32, (1, S), 1)
    tri = (icol <= irow).astype(f32)
    prefix = jnp.dot(tie.astype(f32), tri, preferred_element_type=f32)
    selm = jnp.logical_or(gt, jnp.logical_and(
        tie, prefix <= (K - g).astype(f32)))

    b_col = jax.lax.broadcasted_iota(jnp.int32, (B, 1), 0)
    r_row = jax.lax.broadcasted_iota(jnp.int32, (1, R), 1)
    oh_b = ((r_row // C) == b_col).astype(f32)
    ns = jnp.where(selm, sflat, sflat * f32(0.7))
    msum = jnp.dot(oh_b, ns, precision=_HIGH,
                   preferred_element_type=f32)
    pw = msum * f32(1.0 / C)
    thr = jnp.mean(msum, axis=1, keepdims=True)
    mvals = jnp.where(msum > thr, pw, f32(0.0))
    iota_s = jax.lax.broadcasted_iota(jnp.int32, (B, S), 1)
    mx = jnp.max(mvals, axis=1, keepdims=True)
    ridx = jnp.min(jnp.where(mvals == mx, iota_s, S),
                   axis=1, keepdims=True)

    ai = ridx // H
    aj = ridx - ai * H
    ii = iota_s // H
    jj = iota_s - ii * H
    inv = f32(1.0 / H)
    rel_i = (ii.astype(f32) - ai.astype(f32)) * inv
    rel_j = (jj.astype(f32) - aj.astype(f32)) * inv
    rdist = jnp.sqrt(rel_i * rel_i + rel_j * rel_j)
    rang = (jnp.arctan2(rel_j, rel_i) * f32(1.0 / np.pi) + f32(1.0)) * f32(0.5)
    a = jnp.sum(pw * rdist, axis=1, keepdims=True)
    b = jnp.sum(pw * rang, axis=1, keepdims=True)
    pp = jnp.sum(jnp.where(pw > 0, pw * pw, f32(0.0)), axis=1, keepdims=True)
    pn = jnp.sum(jnp.where(pw < 0, pw * pw, f32(0.0)), axis=1, keepdims=True)
    u = a * w1_ref[0:1, :] + b * w1_ref[1:2, :]
    wvec = jnp.where(u > 0, pp, pn) * u
    gv = jnp.dot(wvec, w2_ref[:, :], precision=_HIGH,
                 preferred_element_type=f32)
    pw_a = jnp.sum(jnp.where(iota_s == ridx, pw, f32(0.0)),
                   axis=1, keepdims=True)
    z = pw_a * gv
    add_ref[:, 0, :] = jnp.where(z >= 0, z, f32(0.2) * z)

    count = jnp.dot(oh_b, selm.astype(f32), preferred_element_type=f32)
    pi = icol // H
    pj = icol - pi * H
    qi = irow // H
    qj = irow - qi * H
    di = jnp.abs(pi - qi)
    dj = jnp.abs(pj - qj)
    t1 = jnp.where(di == 0, f32(2.0), jnp.where(di == 1, f32(1.0), f32(0.0)))
    t2 = jnp.where(dj == 0, f32(2.0), jnp.where(dj == 1, f32(1.0), f32(0.0)))
    m_band = t1 * t2
    y = jnp.dot(count, m_band, precision=_HIGH,
                preferred_element_type=f32)

    irow_f = irow.astype(f32)
    base_row = (S - 1 - irow).astype(f32)
    base_col = (S - 1 - icol).astype(f32)
    iota_col_k = jax.lax.broadcasted_iota(jnp.int32, (K, 1), 0).astype(f32)
    ones_col = jnp.ones((S, 1), f32)
    for bi in range(B):
        y_row = y[bi:bi + 1, :]
        dy = jnp.where(icol == irow, y_row, f32(0.0))
        y_col = jnp.dot(dy, ones_col, precision=_HIGH,
                        preferred_element_type=f32)
        key_row = y_row * f32(S) + base_row
        key_col = y_col * f32(S) + base_col
        rank_col = jnp.sum((key_row > key_col).astype(f32),
                           axis=1, keepdims=True)
        rank_row = jnp.sum((key_col > key_row).astype(f32),
                           axis=0, keepdims=True)
        pos_b = jnp.where(rank_col == irow_f,
                          (icol + 1).astype(f32), f32(0.0))
        pos_row = jnp.sum(pos_b, axis=0, keepdims=True)
        pidx_ref[bi] = pos_row[:, :K].astype(jnp.int32)
        oh_ref[bi] = (rank_row == iota_col_k).astype(f32)


def _stream_body(hid_ref, add_ref, oh_ref, outh_ref, sel_ref):
    outh_ref[0] = hid_ref[0]
    outh_ref[0, 0:1, :] = hid_ref[0, 0:1, :] + add_ref[0]
    sel_ref[0] = jnp.dot(oh_ref[0], hid_ref[0, 1:, :], precision=_HIGH,
                         preferred_element_type=jnp.float32)


def kernel(hidden_states, x, contribution, W1, W2):
    B, C = x.shape[0], x.shape[1]
    S = x.shape[3] - 1
    K = 84
    H = int(np.sqrt(S))
    T, HD = hidden_states.shape[1], hidden_states.shape[2]
    score = x[:, :, 0, 1:].reshape(B * C, S)

    sel_fn = functools.partial(_select_body, B=B, C=C, S=S, K=K, H=H)
    add_vec, pidx, onehot = pl.pallas_call(
        sel_fn,
        grid=(1,),
        in_specs=[
            pl.BlockSpec((B * C, S), lambda i: (0, 0)),
            pl.BlockSpec((2, 512), lambda i: (0, 0)),
            pl.BlockSpec((512, HD), lambda i: (0, 0)),
        ],
        out_specs=[
            pl.BlockSpec((B, 1, HD), lambda i: (0, 0, 0)),
            pl.BlockSpec((B, 1, K), lambda i: (0, 0, 0)),
            pl.BlockSpec((B, K, S), lambda i: (0, 0, 0)),
        ],
        out_shape=[
            jax.ShapeDtypeStruct((B, 1, HD), jnp.float32),
            jax.ShapeDtypeStruct((B, 1, K), jnp.int32),
            jax.ShapeDtypeStruct((B, K, S), jnp.float32),
        ],
    )(score, W1, W2)

    outh, sel = pl.pallas_call(
        _stream_body,
        grid=(B,),
        in_specs=[
            pl.BlockSpec((1, T, HD), lambda b: (b, 0, 0)),
            pl.BlockSpec((1, 1, HD), lambda b: (b, 0, 0)),
            pl.BlockSpec((1, K, S), lambda b: (b, 0, 0)),
        ],
        out_specs=[
            pl.BlockSpec((1, T, HD), lambda b: (b, 0, 0)),
            pl.BlockSpec((1, K, HD), lambda b: (b, 0, 0)),
        ],
        out_shape=[
            jax.ShapeDtypeStruct((B, T, HD), jnp.float32),
            jax.ShapeDtypeStruct((B, K, HD), jnp.float32),
        ],
    )(hidden_states, add_vec, onehot)
    return outh, sel, pidx.reshape(B, K)


# speedup vs baseline: 3.5016x; 1.0090x over previous
import functools
import numpy as np
import jax
import jax.numpy as jnp
from jax.experimental import pallas as pl
from jax.experimental.pallas import tpu as pltpu

_HIGH = jax.lax.Precision.HIGHEST


def _select_compute(score_ref, w1_ref, w2_ref, add_ref, pidx_ref, oh_ref,
                    B, C, S, K, H):
    f32 = jnp.float32
    R = B * C
    sflat = score_ref[...]

    sb = jax.lax.bitcast_convert_type(sflat, jnp.int32)
    keys = sb ^ (jax.lax.shift_right_arithmetic(sb, 31) &
                 jnp.int32(0x7FFFFFFF))
    rmin = jnp.min(keys, axis=1, keepdims=True)
    rmax = jnp.max(keys, axis=1, keepdims=True)
    g0 = jnp.sum((keys >= 0).astype(jnp.int32), axis=1, keepdims=True)
    pos_side = g0 >= K
    lo0 = jnp.where(pos_side, jnp.maximum(rmin, 0), rmin)
    hi0 = jnp.where(pos_side, rmax, jnp.minimum(rmax, -1))

    def bstep(_, carry):
        lo, hi = carry
        mid = lo + jax.lax.shift_right_arithmetic(hi - lo, 1)
        cnt = jnp.sum((keys > mid).astype(jnp.int32), axis=1, keepdims=True)
        ge = cnt >= K
        return (jnp.where(ge, mid + 1, lo), jnp.where(ge, hi, mid))

    t, _ = jax.lax.fori_loop(0, 31, bstep, (lo0, hi0))
    gt = keys > t
    g = jnp.sum(gt.astype(jnp.int32), axis=1, keepdims=True)
    tie = keys == t
    icol = jax.lax.broadcasted_iota(jnp.int32, (S, 1), 0)
    irow = jax.lax.broadcasted_iota(jnp.int32, (1, S), 1)
    tri = (icol <= irow).astype(f32)
    prefix = jnp.dot(tie.astype(f32), tri, preferred_element_type=f32)
    selm = jnp.logical_or(gt, jnp.logical_and(
        tie, prefix <= (K - g).astype(f32)))

    b_col = jax.lax.broadcasted_iota(jnp.int32, (B, 1), 0)
    r_row = jax.lax.broadcasted_iota(jnp.int32, (1, R), 1)
    oh_b = ((r_row // C) == b_col).astype(f32)
    ns = jnp.where(selm, sflat, sflat * f32(0.7))
    msum = jnp.dot(oh_b, ns, precision=_HIGH,
                   preferred_element_type=f32)
    pw = msum * f32(1.0 / C)
    thr = jnp.mean(msum, axis=1, keepdims=True)
    mvals = jnp.where(msum > thr, pw, f32(0.0))
    iota_s = jax.lax.broadcasted_iota(jnp.int32, (B, S), 1)
    mx = jnp.max(mvals, axis=1, keepdims=True)
    ridx = jnp.min(jnp.where(mvals == mx, iota_s, S),
                   axis=1, keepdims=True)

    ai = ridx // H
    aj = ridx - ai * H
    ii = iota_s // H
    jj = iota_s - ii * H
    inv = f32(1.0 / H)
    rel_i = (ii.astype(f32) - ai.astype(f32)) * inv
    rel_j = (jj.astype(f32) - aj.astype(f32)) * inv
    rdist = jnp.sqrt(rel_i * rel_i + rel_j * rel_j)
    rang = (jnp.arctan2(rel_j, rel_i) * f32(1.0 / np.pi) + f32(1.0)) * f32(0.5)
    a = jnp.sum(pw * rdist, axis=1, keepdims=True)
    b = jnp.sum(pw * rang, axis=1, keepdims=True)
    pp = jnp.sum(jnp.where(pw > 0, pw * pw, f32(0.0)), axis=1, keepdims=True)
    pn = jnp.sum(jnp.where(pw < 0, pw * pw, f32(0.0)), axis=1, keepdims=True)
    u = a * w1_ref[0:1, :] + b * w1_ref[1:2, :]
    wvec = jnp.where(u > 0, pp, pn) * u
    gv = jnp.dot(wvec, w2_ref[:, :], precision=_HIGH,
                 preferred_element_type=f32)
    pw_a = jnp.sum(jnp.where(iota_s == ridx, pw, f32(0.0)),
                   axis=1, keepdims=True)
    z = pw_a * gv
    add_ref[:, 0, :] = jnp.where(z >= 0, z, f32(0.2) * z)

    count = jnp.dot(oh_b, selm.astype(f32), preferred_element_type=f32)
    pi = icol // H
    pj = icol - pi * H
    qi = irow // H
    qj = irow - qi * H
    di = jnp.abs(pi - qi)
    dj = jnp.abs(pj - qj)
    t1 = jnp.where(di == 0, f32(2.0), jnp.where(di == 1, f32(1.0), f32(0.0)))
    t2 = jnp.where(dj == 0, f32(2.0), jnp.where(dj == 1, f32(1.0), f32(0.0)))
    m_band = t1 * t2
    y = jnp.dot(count, m_band, precision=_HIGH,
                preferred_element_type=f32)

    irow_f = irow.astype(f32)
    base_row = (S - 1 - irow).astype(f32)
    base_col = (S - 1 - icol).astype(f32)
    iota_col_k = jax.lax.broadcasted_iota(jnp.int32, (K, 1), 0).astype(f32)
    ones_col = jnp.ones((S, 1), f32)
    for bi in range(B):
        y_row = y[bi:bi + 1, :]
        dy = jnp.where(icol == irow, y_row, f32(0.0))
        y_col = jnp.dot(dy, ones_col, precision=_HIGH,
                        preferred_element_type=f32)
        key_row = y_row * f32(S) + base_row
        key_col = y_col * f32(S) + base_col
        rank_col = jnp.sum((key_row > key_col).astype(f32),
                           axis=1, keepdims=True)
        rank_row = jnp.sum((key_col > key_row).astype(f32),
                           axis=0, keepdims=True)
        pos_b = jnp.where(rank_col == irow_f,
                          (icol + 1).astype(f32), f32(0.0))
        pos_row = jnp.sum(pos_b, axis=0, keepdims=True)
        pidx_ref[bi] = pos_row[:, :K].astype(jnp.int32)
        oh_ref[bi] = (rank_row == iota_col_k).astype(f32)


def _fused_body(score_ref, w1_ref, w2_ref, hid_ref,
                outh_ref, sel_ref, add_ref, pidx_ref, oh_scr,
                *, B, C, S, K, H):
    b = pl.program_id(0)

    @pl.when(b == 0)
    def _():
        _select_compute(score_ref, w1_ref, w2_ref, add_ref, pidx_ref, oh_scr,
                        B, C, S, K, H)

    @pl.when(b > 0)
    def _():
        bi = b - 1
        outh_ref[0] = hid_ref[0]
        outh_ref[0, 0:1, :] = (hid_ref[0, 0:1, :] +
                               add_ref[pl.ds(bi, 1), 0, :])
        oh = oh_scr[pl.ds(bi, 1), :, :].reshape(K, S)
        sel_ref[0] = jnp.dot(oh, hid_ref[0, 1:, :], precision=_HIGH,
                             preferred_element_type=jnp.float32)


def kernel(hidden_states, x, contribution, W1, W2):
    B, C = x.shape[0], x.shape[1]
    S = x.shape[3] - 1
    K = 84
    H = int(np.sqrt(S))
    T, HD = hidden_states.shape[1], hidden_states.shape[2]
    score = x[:, :, 0, 1:].reshape(B * C, S)

    body = functools.partial(_fused_body, B=B, C=C, S=S, K=K, H=H)
    outh, sel, add_vec, pidx = pl.pallas_call(
        body,
        grid=(B + 1,),
        in_specs=[
            pl.BlockSpec((B * C, S), lambda b: (0, 0)),
            pl.BlockSpec((2, 512), lambda b: (0, 0)),
            pl.BlockSpec((512, HD), lambda b: (0, 0)),
            pl.BlockSpec((1, T, HD),
                         lambda b: (jnp.maximum(b - 1, 0), 0, 0)),
        ],
        out_specs=[
            pl.BlockSpec((1, T, HD),
                         lambda b: (jnp.maximum(b - 1, 0), 0, 0)),
            pl.BlockSpec((1, K, HD),
                         lambda b: (jnp.maximum(b - 1, 0), 0, 0)),
            pl.BlockSpec((B, 1, HD), lambda b: (0, 0, 0)),
            pl.BlockSpec((B, 1, K), lambda b: (0, 0, 0)),
        ],
        out_shape=[
            jax.ShapeDtypeStruct((B, T, HD), jnp.float32),
            jax.ShapeDtypeStruct((B, K, HD), jnp.float32),
            jax.ShapeDtypeStruct((B, 1, HD), jnp.float32),
            jax.ShapeDtypeStruct((B, 1, K), jnp.int32),
        ],
        scratch_shapes=[pltpu.VMEM((B, K, S), jnp.float32)],
    )(score, W1, W2, hidden_states)
    return outh, sel, pidx.reshape(B, K)


# speedup vs baseline: 3.9233x; 1.1204x over previous
import functools
import numpy as np
import jax
import jax.numpy as jnp
from jax.experimental import pallas as pl
from jax.experimental.pallas import tpu as pltpu

_HIGH = jax.lax.Precision.HIGHEST


def _select_compute(score_ref, w1_ref, w2_ref, add_ref, pidx_ref, oh_ref,
                    B, C, S, K, H):
    f32 = jnp.float32
    R = B * C
    sflat = score_ref[...]

    sb = jax.lax.bitcast_convert_type(sflat, jnp.int32)
    keys = sb ^ (jax.lax.shift_right_arithmetic(sb, 31) &
                 jnp.int32(0x7FFFFFFF))
    rmin = jnp.min(keys, axis=1, keepdims=True)
    rmax = jnp.max(keys, axis=1, keepdims=True)
    g0 = jnp.sum((keys >= 0).astype(jnp.int32), axis=1, keepdims=True)
    pos_side = g0 >= K
    lo0 = jnp.where(pos_side, jnp.maximum(rmin, 0), rmin)
    hi0 = jnp.where(pos_side, rmax, jnp.minimum(rmax, -1))

    def bstep(_, carry):
        lo, hi = carry
        mid = lo + jax.lax.shift_right_arithmetic(hi - lo, 1)
        cnt = jnp.sum((keys > mid).astype(jnp.int32), axis=1, keepdims=True)
        ge = cnt >= K
        return (jnp.where(ge, mid + 1, lo), jnp.where(ge, hi, mid))

    t, _ = jax.lax.fori_loop(0, 31, bstep, (lo0, hi0))
    gt = keys > t
    g = jnp.sum(gt.astype(jnp.int32), axis=1, keepdims=True)
    tie = keys == t
    icol = jax.lax.broadcasted_iota(jnp.int32, (S, 1), 0)
    irow = jax.lax.broadcasted_iota(jnp.int32, (1, S), 1)
    tri = (icol <= irow).astype(f32)
    prefix = jnp.dot(tie.astype(f32), tri, preferred_element_type=f32)
    selm = jnp.logical_or(gt, jnp.logical_and(
        tie, prefix <= (K - g).astype(f32)))

    b_col = jax.lax.broadcasted_iota(jnp.int32, (B, 1), 0)
    r_row = jax.lax.broadcasted_iota(jnp.int32, (1, R), 1)
    oh_b = ((r_row // C) == b_col).astype(f32)
    ns = jnp.where(selm, sflat, sflat * f32(0.7))
    msum = jnp.dot(oh_b, ns, precision=_HIGH,
                   preferred_element_type=f32)
    pw = msum * f32(1.0 / C)
    thr = jnp.mean(msum, axis=1, keepdims=True)
    mvals = jnp.where(msum > thr, pw, f32(0.0))
    iota_s = jax.lax.broadcasted_iota(jnp.int32, (B, S), 1)
    mx = jnp.max(mvals, axis=1, keepdims=True)
    ridx = jnp.min(jnp.where(mvals == mx, iota_s, S),
                   axis=1, keepdims=True)

    ai = ridx // H
    aj = ridx - ai * H
    ii = iota_s // H
    jj = iota_s - ii * H
    inv = f32(1.0 / H)
    rel_i = (ii.astype(f32) - ai.astype(f32)) * inv
    rel_j = (jj.astype(f32) - aj.astype(f32)) * inv
    rdist = jnp.sqrt(rel_i * rel_i + rel_j * rel_j)
    rang = (jnp.arctan2(rel_j, rel_i) * f32(1.0 / np.pi) + f32(1.0)) * f32(0.5)
    a = jnp.sum(pw * rdist, axis=1, keepdims=True)
    b = jnp.sum(pw * rang, axis=1, keepdims=True)
    pp = jnp.sum(jnp.where(pw > 0, pw * pw, f32(0.0)), axis=1, keepdims=True)
    pn = jnp.sum(jnp.where(pw < 0, pw * pw, f32(0.0)), axis=1, keepdims=True)
    u = a * w1_ref[0:1, :] + b * w1_ref[1:2, :]
    wvec = jnp.where(u > 0, pp, pn) * u
    gv = jnp.dot(wvec, w2_ref[:, :], precision=_HIGH,
                 preferred_element_type=f32)
    pw_a = jnp.sum(jnp.where(iota_s == ridx, pw, f32(0.0)),
                   axis=1, keepdims=True)
    z = pw_a * gv
    add_ref[:, 0, :] = jnp.where(z >= 0, z, f32(0.2) * z)

    count = jnp.dot(oh_b, selm.astype(f32), preferred_element_type=f32)
    pi = icol // H
    pj = icol - pi * H
    qi = irow // H
    qj = irow - qi * H
    di = jnp.abs(pi - qi)
    dj = jnp.abs(pj - qj)
    t1 = jnp.where(di == 0, f32(2.0), jnp.where(di == 1, f32(1.0), f32(0.0)))
    t2 = jnp.where(dj == 0, f32(2.0), jnp.where(dj == 1, f32(1.0), f32(0.0)))
    m_band = t1 * t2
    y = jnp.dot(count, m_band, precision=_HIGH,
                preferred_element_type=f32)

    key8 = y * f32(S) + (S - 1 - irow).astype(f32)
    rank8 = jnp.sum((key8[:, :, None] > key8[:, None, :]).astype(f32),
                    axis=1)
    iota_q3 = jax.lax.broadcasted_iota(jnp.int32, (1, S, 1), 1)
    iota_r3 = jax.lax.broadcasted_iota(jnp.int32, (1, 1, S), 2).astype(f32)
    pos = jnp.sum(jnp.where(rank8[:, :, None] == iota_r3,
                            (iota_q3 + 1).astype(f32), f32(0.0)),
                  axis=1)
    pidx_ref[:, 0, :] = pos[:, :K].astype(jnp.int32)
    iota_k3 = jax.lax.broadcasted_iota(jnp.int32, (1, K, 1), 1).astype(f32)
    oh_ref[...] = (rank8[:, None, :] == iota_k3).astype(f32)


def _fused_body(score_ref, w1_ref, w2_ref, hid_ref,
                outh_ref, sel_ref, add_ref, pidx_ref, oh_scr,
                *, B, C, S, K, H):
    b = pl.program_id(0)

    @pl.when(b == 0)
    def _():
        _select_compute(score_ref, w1_ref, w2_ref, add_ref, pidx_ref, oh_scr,
                        B, C, S, K, H)

    @pl.when(b > 0)
    def _():
        bi = b - 1
        outh_ref[0] = hid_ref[0]
        outh_ref[0, 0:1, :] = (hid_ref[0, 0:1, :] +
                               add_ref[pl.ds(bi, 1), 0, :])
        oh = oh_scr[pl.ds(bi, 1), :, :].reshape(K, S)
        sel_ref[0] = jnp.dot(oh, hid_ref[0, 1:, :], precision=_HIGH,
                             preferred_element_type=jnp.float32)


def kernel(hidden_states, x, contribution, W1, W2):
    B, C = x.shape[0], x.shape[1]
    S = x.shape[3] - 1
    K = 84
    H = int(np.sqrt(S))
    T, HD = hidden_states.shape[1], hidden_states.shape[2]
    score = x[:, :, 0, 1:].reshape(B * C, S)

    body = functools.partial(_fused_body, B=B, C=C, S=S, K=K, H=H)
    outh, sel, add_vec, pidx = pl.pallas_call(
        body,
        grid=(B + 1,),
        in_specs=[
            pl.BlockSpec((B * C, S), lambda b: (0, 0)),
            pl.BlockSpec((2, 512), lambda b: (0, 0)),
            pl.BlockSpec((512, HD), lambda b: (0, 0)),
            pl.BlockSpec((1, T, HD),
                         lambda b: (jnp.maximum(b - 1, 0), 0, 0)),
        ],
        out_specs=[
            pl.BlockSpec((1, T, HD),
                         lambda b: (jnp.maximum(b - 1, 0), 0, 0)),
            pl.BlockSpec((1, K, HD),
                         lambda b: (jnp.maximum(b - 1, 0), 0, 0)),
            pl.BlockSpec((B, 1, HD), lambda b: (0, 0, 0)),
            pl.BlockSpec((B, 1, K), lambda b: (0, 0, 0)),
        ],
        out_shape=[
            jax.ShapeDtypeStruct((B, T, HD), jnp.float32),
            jax.ShapeDtypeStruct((B, K, HD), jnp.float32),
            jax.ShapeDtypeStruct((B, 1, HD), jnp.float32),
            jax.ShapeDtypeStruct((B, 1, K), jnp.int32),
        ],
        scratch_shapes=[pltpu.VMEM((B, K, S), jnp.float32)],
    )(score, W1, W2, hidden_states)
    return outh, sel, pidx.reshape(B, K)
